# baseline (speedup 1.0000x reference)
import functools

import jax
import jax.numpy as jnp
from jax import lax
from jax.experimental import pallas as pl
from jax.experimental.pallas import tpu as pltpu
from jax.experimental.pallas import tpu_sc as plsc

_LANES = 16


def _make_sc_add(B, S, D):
    NC, NS = 2, 16
    NW = NC * NS
    rows_per_w = S // NW
    CH = 16
    n_chunks = rows_per_w // CH
    chunk = CH * D

    mesh = plsc.VectorSubcoreMesh(core_axis_name="c", subcore_axis_name="s")

    @functools.partial(
        pl.kernel,
        out_type=jax.ShapeDtypeStruct((B * S * D,), jnp.float32),
        mesh=mesh,
        scratch_types=[
            pltpu.VMEM((chunk,), jnp.float32),
            pltpu.VMEM((chunk,), jnp.float32),
        ],
    )
    def sc_add(x_hbm, pe_hbm, out_hbm, pe_v, x_v):
        wid = lax.axis_index("s") * NC + lax.axis_index("c")
        s0 = wid * rows_per_w

        def chunk_body(c, carry):
            base = (s0 + c * CH) * D
            pltpu.sync_copy(pe_hbm.at[pl.ds(base, chunk)], pe_v)

            def batch_body(b, carry2):
                x_off = b * S * D + base
                pltpu.sync_copy(x_hbm.at[pl.ds(x_off, chunk)], x_v)

                @plsc.parallel_loop(0, chunk // _LANES, unroll=8)
                def _add(i):
                    sl = pl.ds(i * _LANES, _LANES)
                    x_v[sl] = x_v[sl] + pe_v[sl]

                pltpu.sync_copy(x_v, out_hbm.at[pl.ds(x_off, chunk)])
                return carry2

            return lax.fori_loop(0, B, batch_body, carry)

        lax.fori_loop(0, n_chunks, chunk_body, 0)

    return sc_add


def _add_body(x_ref, pe_ref, o_ref):
    o_ref[...] = x_ref[...] + pe_ref[...]


def _tc_kernel(x, pos_embedding):
    B, S, D = x.shape
    BS = 2048
    return pl.pallas_call(
        _add_body,
        grid=(S // BS, B),
        in_specs=[
            pl.BlockSpec((1, BS, D), lambda s, b: (b, s, 0)),
            pl.BlockSpec((BS, D), lambda s, b: (s, 0)),
        ],
        out_specs=pl.BlockSpec((1, BS, D), lambda s, b: (b, s, 0)),
        out_shape=jax.ShapeDtypeStruct((B, S, D), x.dtype),
    )(x, pos_embedding)


def kernel(x, pos_embedding):
    B, S, D = x.shape
    out = _make_sc_add(B, S, D)(x.reshape(-1), pos_embedding.reshape(-1))
    return out.reshape(B, S, D)


# speedup vs baseline: 1.2771x; 1.2771x over previous
import functools

import jax
import jax.numpy as jnp
from jax import lax
from jax.experimental import pallas as pl
from jax.experimental.pallas import tpu as pltpu
from jax.experimental.pallas import tpu_sc as plsc

_LANES = 16


def _make_sc_add(B, S, D):
    NC, NS = 2, 16
    NW = NC * NS
    rows_per_w = S // NW
    CH = 16
    n_chunks = rows_per_w // CH
    chunk = CH * D
    n_k = n_chunks * B

    mesh = plsc.VectorSubcoreMesh(core_axis_name="c", subcore_axis_name="s")

    @functools.partial(
        pl.kernel,
        out_type=jax.ShapeDtypeStruct((B * S * D,), jnp.float32),
        mesh=mesh,
        scratch_types=[
            [pltpu.VMEM((chunk,), jnp.float32) for _ in range(2)],
            [pltpu.VMEM((chunk,), jnp.float32) for _ in range(2)],
            [pltpu.VMEM((chunk,), jnp.float32) for _ in range(2)],
            [pltpu.SemaphoreType.DMA for _ in range(2)],
            [pltpu.SemaphoreType.DMA for _ in range(2)],
            [pltpu.SemaphoreType.DMA for _ in range(2)],
        ],
    )
    def sc_add(x_hbm, pe_hbm, out_hbm, x_v, o_v, pe_v, sx, so, sp):
        wid = lax.axis_index("s") * NC + lax.axis_index("c")
        base0 = wid * rows_per_w * D

        def x_off(k):
            c = k // B
            b = k % B
            return b * S * D + base0 + c * chunk

        def pe_off(c):
            return base0 + c * chunk

        pltpu.async_copy(x_hbm.at[pl.ds(x_off(0), chunk)], x_v[0], sx[0])
        pltpu.async_copy(x_hbm.at[pl.ds(x_off(1), chunk)], x_v[1], sx[1])
        pltpu.async_copy(pe_hbm.at[pl.ds(pe_off(0), chunk)], pe_v[0], sp[0])
        pltpu.async_copy(pe_hbm.at[pl.ds(pe_off(1), chunk)], pe_v[1], sp[1])

        def q_body(q, carry):
            for cc in range(2):
                c = 2 * q + cc
                pltpu.make_async_copy(
                    pe_hbm.at[pl.ds(pe_off(c), chunk)], pe_v[cc], sp[cc]
                ).wait()
                for bb in range(B):
                    k = c * B + bb
                    j = (cc * B + bb) % 2
                    @pl.when(k >= 2)
                    def _():
                        pltpu.make_async_copy(
                            o_v[j], out_hbm.at[pl.ds(x_off(k - 2), chunk)],
                            so[j],
                        ).wait()
                    pltpu.make_async_copy(
                        x_hbm.at[pl.ds(x_off(k), chunk)], x_v[j], sx[j]
                    ).wait()

                    @plsc.parallel_loop(0, chunk // _LANES, unroll=8)
                    def _add(i):
                        sl = pl.ds(i * _LANES, _LANES)
                        o_v[j][sl] = x_v[j][sl] + pe_v[cc][sl]

                    pltpu.async_copy(
                        o_v[j], out_hbm.at[pl.ds(x_off(k), chunk)], so[j]
                    )

                    @pl.when(k + 2 < n_k)
                    def _():
                        pltpu.async_copy(
                            x_hbm.at[pl.ds(x_off(k + 2), chunk)], x_v[j],
                            sx[j],
                        )
                @pl.when(c + 2 < n_chunks)
                def _():
                    pltpu.async_copy(
                        pe_hbm.at[pl.ds(pe_off(c + 2), chunk)], pe_v[cc],
                        sp[cc],
                    )
            return carry

        lax.fori_loop(0, n_chunks // 2, q_body, 0)

        for j in range(2):
            k = n_k - 2 + j
            pltpu.make_async_copy(
                o_v[j], out_hbm.at[pl.ds(x_off(k), chunk)], so[j]
            ).wait()

    return sc_add


def _add_body(x_ref, pe_ref, o_ref):
    o_ref[...] = x_ref[...] + pe_ref[...]


def _tc_kernel(x, pos_embedding):
    B, S, D = x.shape
    BS = 2048
    return pl.pallas_call(
        _add_body,
        grid=(S // BS, B),
        in_specs=[
            pl.BlockSpec((1, BS, D), lambda s, b: (b, s, 0)),
            pl.BlockSpec((BS, D), lambda s, b: (s, 0)),
        ],
        out_specs=pl.BlockSpec((1, BS, D), lambda s, b: (b, s, 0)),
        out_shape=jax.ShapeDtypeStruct((B, S, D), x.dtype),
    )(x, pos_embedding)


def kernel(x, pos_embedding):
    B, S, D = x.shape
    out = _make_sc_add(B, S, D)(x.reshape(-1), pos_embedding.reshape(-1))
    return out.reshape(B, S, D)
